# baseline (speedup 1.0000x reference)
import functools

import jax
import jax.numpy as jnp
from jax import lax
from jax.experimental import pallas as pl
from jax.experimental.pallas import tpu as pltpu
from jax.experimental.pallas import tpu_sc as plsc

B = 16384
D = 32
NC = 2
NS = 16
NW = NC * NS
BPW = B // NW
CHUNK = 128
NCHUNK = BPW // CHUNK


def _sc_body(uidx_hbm, iidx_hbm, utab_hbm, itab_hbm, out_hbm,
             uidx_v, iidx_v, urows_v, irows_v, out_v, usem, isem):
    c = lax.axis_index("c")
    s = lax.axis_index("s")
    wid = s * NC + c
    base = wid * BPW

    for j in range(NCHUNK):
        pltpu.sync_copy(uidx_hbm.at[pl.ds(base + j * CHUNK, CHUNK)],
                        uidx_v.at[j])
        pltpu.sync_copy(iidx_hbm.at[pl.ds(base + j * CHUNK, CHUNK)],
                        iidx_v.at[j])

    copies = []
    for j in range(NCHUNK):
        copies.append(pltpu.async_copy(
            utab_hbm.at[uidx_v.at[j]],
            urows_v.at[pl.ds(j * CHUNK, CHUNK)], usem))
        copies.append(pltpu.async_copy(
            itab_hbm.at[iidx_v.at[j]],
            irows_v.at[pl.ds(j * CHUNK, CHUNK)], isem))
    for cp in copies:
        cp.wait()

    lane = lax.iota(jnp.int32, 16)

    @pl.loop(0, BPW // 16)
    def _chunk(ci):
        rows = ci * 16 + lane
        acc = jnp.zeros((16,), jnp.float32)
        for d in range(D):
            col = jnp.full((16,), d, jnp.int32)
            u = plsc.load_gather(urows_v, [rows, col])
            v = plsc.load_gather(irows_v, [rows, col])
            acc = acc + u * v
        out_v[pl.ds(ci * 16, 16)] = acc

    pltpu.sync_copy(out_v, out_hbm.at[pl.ds(base, BPW)])


@jax.jit
def _score(user_indices, item_indices, user_table, item_table):
    mesh = plsc.VectorSubcoreMesh(core_axis_name="c", subcore_axis_name="s",
                                  num_cores=NC, num_subcores=NS)
    run = pl.kernel(
        _sc_body,
        out_type=jax.ShapeDtypeStruct((B,), jnp.float32),
        mesh=mesh,
        compiler_params=pltpu.CompilerParams(needs_layout_passes=False,
                                             use_tc_tiling_on_sc=False),
        scratch_types=[
            pltpu.VMEM((NCHUNK, CHUNK), jnp.int32),
            pltpu.VMEM((NCHUNK, CHUNK), jnp.int32),
            pltpu.VMEM((BPW, D), jnp.float32),
            pltpu.VMEM((BPW, D), jnp.float32),
            pltpu.VMEM((BPW,), jnp.float32),
            pltpu.SemaphoreType.DMA,
            pltpu.SemaphoreType.DMA,
        ],
    )
    return run(user_indices, item_indices, user_table, item_table)


def kernel(user_indices, item_indices, user_table, item_table):
    return _score(user_indices.astype(jnp.int32),
                  item_indices.astype(jnp.int32),
                  user_table, item_table)


# speedup vs baseline: 3.9818x; 3.9818x over previous
import functools

import jax
import jax.numpy as jnp
from jax import lax
from jax.experimental import pallas as pl
from jax.experimental.pallas import tpu as pltpu
from jax.experimental.pallas import tpu_sc as plsc

B = 16384
D = 32
NC = 2
NS = 16
NW = NC * NS
BPW = B // NW
NCHUNK = BPW // 16
K = 8


def _sc_body(uidx_hbm, iidx_hbm, utab_hbm, itab_hbm, out_hbm,
             uvm, ivm, ovm, ubuf, ibuf, usems, isems):
    c = lax.axis_index("c")
    s = lax.axis_index("s")
    wid = s * NC + c
    base = wid * BPW

    pltpu.sync_copy(uidx_hbm.at[pl.ds(base, BPW)], uvm)
    pltpu.sync_copy(iidx_hbm.at[pl.ds(base, BPW)], ivm)

    lanes = lax.iota(jnp.int32, 16)
    rows0 = lanes
    rows1 = lanes + 16

    def issue(u, i, slot):
        uoff = pl.multiple_of((u >> 7) << 7, 128)
        ioff = pl.multiple_of((i >> 7) << 7, 128)
        pltpu.async_copy(utab_hbm.at[:, pl.ds(uoff, 128)],
                         ubuf.at[slot], usems.at[slot])
        pltpu.async_copy(itab_hbm.at[:, pl.ds(ioff, 128)],
                         ibuf.at[slot], isems.at[slot])

    def compute(u, i, slot):
        pltpu.make_async_copy(utab_hbm.at[:, pl.ds(0, 128)],
                              ubuf.at[slot], usems.at[slot]).wait()
        pltpu.make_async_copy(itab_hbm.at[:, pl.ds(0, 128)],
                              ibuf.at[slot], isems.at[slot]).wait()
        slotv = jnp.full((16,), slot, jnp.int32)
        ucol = jnp.full((16,), u & 127, jnp.int32)
        icol = jnp.full((16,), i & 127, jnp.int32)
        u0 = plsc.load_gather(ubuf, [slotv, rows0, ucol])
        u1 = plsc.load_gather(ubuf, [slotv, rows1, ucol])
        i0 = plsc.load_gather(ibuf, [slotv, rows0, icol])
        i1 = plsc.load_gather(ibuf, [slotv, rows1, icol])
        return jnp.sum(u0 * i0 + u1 * i1)

    uvec0 = uvm[pl.ds(0, 16)]
    ivec0 = ivm[pl.ds(0, 16)]
    for k in range(K):
        issue(uvec0[k], ivec0[k], k)

    @pl.loop(0, NCHUNK)
    def _chunk(ci):
        uvec = uvm[pl.ds(ci * 16, 16)]
        ivec = ivm[pl.ds(ci * 16, 16)]
        acc = jnp.zeros((16,), jnp.float32)
        for k in range(16):
            slot = k & (K - 1)
            s_k = compute(uvec[k], ivec[k], slot)
            acc = jnp.where(lanes == k, jnp.full((16,), s_k, jnp.float32),
                            acc)
            if k < 16 - K:
                issue(uvec[k + K], ivec[k + K], slot)
            else:
                @pl.when(ci < NCHUNK - 1)
                def _prefetch_next():
                    nuvec = uvm[pl.ds((ci + 1) * 16, 16)]
                    nivec = ivm[pl.ds((ci + 1) * 16, 16)]
                    issue(nuvec[k + K - 16], nivec[k + K - 16], slot)
        ovm[pl.ds(ci * 16, 16)] = acc

    pltpu.sync_copy(ovm, out_hbm.at[pl.ds(base, BPW)])


@jax.jit
def _score(user_indices, item_indices, user_table_t, item_table_t):
    mesh = plsc.VectorSubcoreMesh(core_axis_name="c", subcore_axis_name="s",
                                  num_cores=NC, num_subcores=NS)
    run = pl.kernel(
        _sc_body,
        out_type=jax.ShapeDtypeStruct((B,), jnp.float32),
        mesh=mesh,
        compiler_params=pltpu.CompilerParams(needs_layout_passes=False),
        scratch_types=[
            pltpu.VMEM((BPW,), jnp.int32),
            pltpu.VMEM((BPW,), jnp.int32),
            pltpu.VMEM((BPW,), jnp.float32),
            pltpu.VMEM((K, D, 128), jnp.float32),
            pltpu.VMEM((K, D, 128), jnp.float32),
            pltpu.SemaphoreType.DMA((K,)),
            pltpu.SemaphoreType.DMA((K,)),
        ],
    )
    return run(user_indices, item_indices, user_table_t, item_table_t)


def kernel(user_indices, item_indices, user_table, item_table):
    return _score(user_indices.astype(jnp.int32),
                  item_indices.astype(jnp.int32),
                  user_table.T, item_table.T)


# speedup vs baseline: 3.9837x; 1.0005x over previous
import jax
import jax.numpy as jnp
from jax import lax
from jax.experimental import pallas as pl
from jax.experimental.pallas import tpu as pltpu
from jax.experimental.pallas import tpu_sc as plsc

B = 16384
D = 32
NC = 2
NS = 16
NW = NC * NS
BPW = B // NW
NCHUNK = BPW // 16
K = 8


def _sc_body(uidx_hbm, iidx_hbm, utab_hbm, itab_hbm, out_hbm,
             uvm, ivm, ovm, ubuf, ibuf, usems, isems):
    c = lax.axis_index("c")
    s = lax.axis_index("s")
    wid = s * NC + c
    base = wid * BPW

    pltpu.sync_copy(uidx_hbm.at[pl.ds(base, BPW)], uvm)
    pltpu.sync_copy(iidx_hbm.at[pl.ds(base, BPW)], ivm)

    lanes = lax.iota(jnp.int32, 16)
    rows0 = lanes
    rows1 = lanes + 16

    def issue(u, i, slot):
        uoff = pl.multiple_of((u >> 7) << 7, 128)
        ioff = pl.multiple_of((i >> 7) << 7, 128)
        pltpu.async_copy(utab_hbm.at[:, pl.ds(uoff, 128)],
                         ubuf.at[slot], usems.at[slot])
        pltpu.async_copy(itab_hbm.at[:, pl.ds(ioff, 128)],
                         ibuf.at[slot], isems.at[slot])

    def compute(u, i, slot):
        pltpu.make_async_copy(utab_hbm.at[:, pl.ds(0, 128)],
                              ubuf.at[slot], usems.at[slot]).wait()
        pltpu.make_async_copy(itab_hbm.at[:, pl.ds(0, 128)],
                              ibuf.at[slot], isems.at[slot]).wait()
        slotv = jnp.full((16,), slot, jnp.int32)
        ucol = jnp.full((16,), u & 127, jnp.int32)
        icol = jnp.full((16,), i & 127, jnp.int32)
        u0 = plsc.load_gather(ubuf, [slotv, rows0, ucol])
        u1 = plsc.load_gather(ubuf, [slotv, rows1, ucol])
        i0 = plsc.load_gather(ibuf, [slotv, rows0, icol])
        i1 = plsc.load_gather(ibuf, [slotv, rows1, icol])
        return jnp.sum(u0 * i0 + u1 * i1)

    uvec0 = uvm[pl.ds(0, 16)]
    ivec0 = ivm[pl.ds(0, 16)]
    for k in range(K):
        issue(uvec0[k], ivec0[k], k)

    @pl.loop(0, NCHUNK)
    def _chunk(ci):
        uvec = uvm[pl.ds(ci * 16, 16)]
        ivec = ivm[pl.ds(ci * 16, 16)]
        acc = jnp.zeros((16,), jnp.float32)
        for k in range(16):
            slot = k & (K - 1)
            s_k = compute(uvec[k], ivec[k], slot)
            acc = jnp.where(lanes == k, jnp.full((16,), s_k, jnp.float32),
                            acc)
            if k < 16 - K:
                issue(uvec[k + K], ivec[k + K], slot)
            else:
                @pl.when(ci < NCHUNK - 1)
                def _prefetch_next():
                    nuvec = uvm[pl.ds((ci + 1) * 16, 16)]
                    nivec = ivm[pl.ds((ci + 1) * 16, 16)]
                    issue(nuvec[k + K - 16], nivec[k + K - 16], slot)
        ovm[pl.ds(ci * 16, 16)] = acc

    pltpu.sync_copy(ovm, out_hbm.at[pl.ds(base, BPW)])


@jax.jit
def _score(user_indices, item_indices, user_table_t, item_table_t):
    mesh = plsc.VectorSubcoreMesh(core_axis_name="c", subcore_axis_name="s",
                                  num_cores=NC, num_subcores=NS)
    run = pl.kernel(
        _sc_body,
        out_type=jax.ShapeDtypeStruct((B,), jnp.float32),
        mesh=mesh,
        compiler_params=pltpu.CompilerParams(needs_layout_passes=False),
        scratch_types=[
            pltpu.VMEM((BPW,), jnp.int32),
            pltpu.VMEM((BPW,), jnp.int32),
            pltpu.VMEM((BPW,), jnp.float32),
            pltpu.VMEM((K, D, 128), jnp.float32),
            pltpu.VMEM((K, D, 128), jnp.float32),
            pltpu.SemaphoreType.DMA((K,)),
            pltpu.SemaphoreType.DMA((K,)),
        ],
    )
    return run(user_indices, item_indices, user_table_t, item_table_t)


def kernel(user_indices, item_indices, user_table, item_table):
    return _score(user_indices.astype(jnp.int32),
                  item_indices.astype(jnp.int32),
                  user_table.T, item_table.T)
